# Initial kernel scaffold; baseline (speedup 1.0000x reference)
import jax
import jax.numpy as jnp
from jax.experimental import pallas as pl


def kernel(x, edge_index, batch_index, W1, b1, W2, b2, W3, b3, W4, b4, W5, b5, Wlb, blb, Wla, bla, Wo, bo):
    raise NotImplementedError("write your pallas kernel here")


# speedup vs baseline: 16.4869x; 16.4869x over previous
import functools

import jax
import jax.numpy as jnp
from jax import lax
from jax.experimental import pallas as pl
from jax.experimental.pallas import tpu as pltpu
from jax.experimental.pallas import tpu_sc as plsc

N = 10000
E = 320000
G = 64
NC = 2
NS = 16
NW = NC * NS
NPAD = 10240
CH = 128
CPT = (E + NW * CH - 1) // (NW * CH)
EPAD = NW * CPT * CH
RPT = NPAD // NS
RPW = NPAD // NW

BR = 1000
GRID = N // BR

_P = lax.Precision.HIGHEST
_mesh = plsc.VectorSubcoreMesh(core_axis_name="c", subcore_axis_name="s")
_SC_PARAMS = pltpu.CompilerParams(use_tc_tiling_on_sc=False)


def _make_agg(F):
    @functools.partial(
        pl.kernel,
        out_type=jax.ShapeDtypeStruct((NC, NPAD, F), jnp.float32),
        mesh=_mesh,
        compiler_params=_SC_PARAMS,
        scratch_types=[
            pltpu.VMEM((CPT, CH), jnp.int32),
            pltpu.VMEM((CPT, CH), jnp.int32),
            pltpu.VMEM((CH, F), jnp.float32),
            pltpu.VMEM_SHARED((NPAD, F), jnp.float32),
        ],
    )
    def agg(z_hbm, src_hbm, dst_hbm, zero_hbm, out_hbm, src_v, dst_v, rows_v, acc_sh):
        c = lax.axis_index("c")
        s = lax.axis_index("s")
        wid = c * NS + s
        pltpu.sync_copy(zero_hbm.at[pl.ds(s * RPT, RPT)],
                        acc_sh.at[pl.ds(s * RPT, RPT)])
        pltpu.sync_copy(src_hbm.at[wid], src_v)
        pltpu.sync_copy(dst_hbm.at[wid], dst_v)
        plsc.subcore_barrier()

        @pl.loop(0, CPT)
        def _(j):
            pltpu.sync_copy(z_hbm.at[src_v.at[j]], rows_v)
            pltpu.sync_copy(rows_v, acc_sh.at[dst_v.at[j]], add=True)

        plsc.subcore_barrier()
        pltpu.sync_copy(acc_sh.at[pl.ds(s * RPT, RPT)],
                        out_hbm.at[c, pl.ds(s * RPT, RPT)])

    return agg


@functools.partial(
    pl.kernel,
    out_type=(jax.ShapeDtypeStruct((NW, G + 1, 128), jnp.float32),
              jax.ShapeDtypeStruct((NW, G + 1, 128), jnp.float32)),
    mesh=_mesh,
    compiler_params=_SC_PARAMS,
    scratch_types=[
        pltpu.VMEM((RPW, 128), jnp.float32),
        pltpu.VMEM((RPW,), jnp.int32),
        pltpu.VMEM((G + 1, 128), jnp.float32),
        pltpu.VMEM((G + 1, 128), jnp.float32),
    ],
)
def _sc_pool(h_hbm, b_hbm, omax_hbm, osum_hbm, rows_v, bid_v, mx_v, sm_v):
    c = lax.axis_index("c")
    s = lax.axis_index("s")
    wid = c * NS + s
    pltpu.sync_copy(h_hbm.at[pl.ds(wid * RPW, RPW)], rows_v)
    pltpu.sync_copy(b_hbm.at[pl.ds(wid * RPW, RPW)], bid_v)

    @pl.loop(0, G + 1)
    def _(g):
        for k in range(8):
            sl = pl.ds(k * 16, 16)
            mx_v[g, sl] = jnp.full((16,), -jnp.inf, jnp.float32)
            sm_v[g, sl] = jnp.zeros((16,), jnp.float32)

    @pl.loop(0, RPW // 16)
    def _(t):
        vb = bid_v[pl.ds(t * 16, 16)]
        for k in range(16):
            bid = vb[k]
            for j in range(8):
                sl = pl.ds(j * 16, 16)
                v = rows_v[t * 16 + k, sl]
                mx_v[bid, sl] = jnp.maximum(mx_v[bid, sl], v)
                sm_v[bid, sl] = sm_v[bid, sl] + v

    pltpu.sync_copy(mx_v, omax_hbm.at[wid])
    pltpu.sync_copy(sm_v, osum_hbm.at[wid])


def _row_spec(f):
    return pl.BlockSpec((BR, f), lambda i: (i, 0))


def _full_spec(shape):
    return pl.BlockSpec(shape, lambda i: tuple(0 for _ in shape))


def _mm(x, w):
    def body(x_ref, w_ref, o_ref):
        o_ref[...] = jnp.dot(x_ref[...], w_ref[...], precision=_P,
                             preferred_element_type=jnp.float32)
    return pl.pallas_call(
        body, grid=(GRID,),
        in_specs=[_row_spec(x.shape[1]), _full_spec(w.shape)],
        out_specs=_row_spec(w.shape[1]),
        out_shape=jax.ShapeDtypeStruct((N, w.shape[1]), jnp.float32),
    )(x, w)


def _tc_dinv_z1(p0, p1, h1):
    def body(p0_ref, p1_ref, h1_ref, dinv_ref, z1_ref):
        deg = 1.0 + p0_ref[:, 0:1] + p1_ref[:, 0:1]
        dinv = 1.0 / jnp.sqrt(deg)
        dinv_ref[...] = dinv
        z1_ref[...] = dinv * h1_ref[...]
    return pl.pallas_call(
        body, grid=(GRID,),
        in_specs=[_row_spec(16), _row_spec(16), _row_spec(16)],
        out_specs=(_row_spec(1), _row_spec(16)),
        out_shape=(jax.ShapeDtypeStruct((N, 1), jnp.float32),
                   jax.ShapeDtypeStruct((N, 16), jnp.float32)),
    )(p0, p1, h1)


def _tc_layer1(q0, q1, h1, dinv, b1):
    def body(q0_ref, q1_ref, h_ref, d_ref, b_ref, a_ref, z_ref):
        dv = d_ref[...]
        u = dv * (q0_ref[...] + q1_ref[...]) + dv * dv * h_ref[...] + b_ref[0:1, :]
        a = jnp.maximum(u, 0.0)
        a_ref[...] = a
        z_ref[...] = dv * a
    return pl.pallas_call(
        body, grid=(GRID,),
        in_specs=[_row_spec(16), _row_spec(16), _row_spec(16), _row_spec(1),
                  _full_spec((8, 16))],
        out_specs=(_row_spec(16), _row_spec(16)),
        out_shape=(jax.ShapeDtypeStruct((N, 16), jnp.float32),
                   jax.ShapeDtypeStruct((N, 16), jnp.float32)),
    )(q0, q1, h1, dinv, b1)


def _tc_layer(q0, q1, ap, dinv, w, b):
    fin = ap.shape[1]
    fout = w.shape[1]

    def body(q0_ref, q1_ref, a_ref, d_ref, w_ref, b_ref, ao_ref, z_ref):
        dv = d_ref[...]
        u = dv * (q0_ref[...] + q1_ref[...]) + dv * dv * a_ref[...]
        a = jnp.maximum(jnp.dot(u, w_ref[...], precision=_P,
                                preferred_element_type=jnp.float32)
                        + b_ref[0:1, :], 0.0)
        ao_ref[...] = a
        z_ref[...] = dv * a
    return pl.pallas_call(
        body, grid=(GRID,),
        in_specs=[_row_spec(fin), _row_spec(fin), _row_spec(fin), _row_spec(1),
                  _full_spec((fin, fout)), _full_spec((8, fout))],
        out_specs=(_row_spec(fout), _row_spec(fout)),
        out_shape=(jax.ShapeDtypeStruct((N, fout), jnp.float32),
                   jax.ShapeDtypeStruct((N, fout), jnp.float32)),
    )(q0, q1, ap, dinv, w, b)


def _tc_layer5(q0, q1, a4, dinv, w5, b5, wlb, blb):
    def body(q0_ref, q1_ref, a_ref, d_ref, w5_ref, b5_ref, wlb_ref, blb_ref, h_ref):
        dv = d_ref[...]
        u = dv * (q0_ref[...] + q1_ref[...]) + dv * dv * a_ref[...]
        a5 = jnp.maximum(jnp.dot(u, w5_ref[...], precision=_P,
                                 preferred_element_type=jnp.float32)
                         + b5_ref[0:1, :], 0.0)
        h_ref[...] = jnp.dot(a5, wlb_ref[...], precision=_P,
                             preferred_element_type=jnp.float32) + blb_ref[0:1, :]
    return pl.pallas_call(
        body, grid=(GRID,),
        in_specs=[_row_spec(64), _row_spec(64), _row_spec(64), _row_spec(1),
                  _full_spec((64, 128)), _full_spec((8, 128)),
                  _full_spec((128, 128)), _full_spec((8, 128))],
        out_specs=_row_spec(128),
        out_shape=jax.ShapeDtypeStruct((N, 128), jnp.float32),
    )(q0, q1, a4, dinv, w5, b5, wlb, blb)


def _tc_final(pmax, psum, batch2d, wla, bla, wo, bo):
    def body(pm_ref, ps_ref, b_ref, wla_ref, bla_ref, wo_ref, bo_ref, o_ref):
        gm = jnp.max(pm_ref[...], axis=0)[:G]
        gs = jnp.sum(ps_ref[...], axis=0)[:G]
        gids = lax.broadcasted_iota(jnp.int32, (G, N), 0)
        mask = (b_ref[...] == gids)
        counts = jnp.sum(mask.astype(jnp.float32), axis=1)
        gm = jnp.where(counts[:, None] > 0, gm, 0.0)
        gmean = gs / jnp.maximum(counts, 1.0)[:, None]
        hc = jnp.concatenate([gm, gmean], axis=1)
        t = jnp.dot(hc, wla_ref[...], precision=_P,
                    preferred_element_type=jnp.float32) + bla_ref[0:1, :]
        o_ref[...] = jnp.dot(t, wo_ref[...], precision=_P,
                             preferred_element_type=jnp.float32) + bo_ref[0:1, :]
    return pl.pallas_call(
        body,
        out_shape=jax.ShapeDtypeStruct((G, 128), jnp.float32),
    )(pmax, psum, batch2d, wla, bla, wo, bo)


def _bb(b):
    return jnp.broadcast_to(b.reshape(1, -1), (8, b.shape[0]))


@jax.jit
def kernel(x, edge_index, batch_index, W1, b1, W2, b2, W3, b3, W4, b4, W5, b5,
           Wlb, blb, Wla, bla, Wo, bo):
    src = edge_index[0]
    dst = edge_index[1]
    epad = EPAD - E
    srcp = jnp.concatenate([src, jnp.zeros((epad,), jnp.int32)]).reshape(NW, CPT, CH)
    dstp = jnp.concatenate([dst, jnp.full((epad,), NPAD - 1, jnp.int32)]).reshape(NW, CPT, CH)

    zeros16 = jnp.zeros((NPAD, 16), jnp.float32)
    zeros32 = jnp.zeros((NPAD, 32), jnp.float32)
    zeros64 = jnp.zeros((NPAD, 64), jnp.float32)
    ones16 = jnp.ones((N, 16), jnp.float32)

    agg16 = _make_agg(16)
    agg32 = _make_agg(32)
    agg64 = _make_agg(64)

    pdeg = agg16(ones16, srcp, dstp, zeros16)
    h1 = _mm(x, W1)
    dinv, z1 = _tc_dinv_z1(pdeg[0, :N], pdeg[1, :N], h1)

    q = agg16(z1, srcp, dstp, zeros16)
    a1, z2 = _tc_layer1(q[0, :N], q[1, :N], h1, dinv, _bb(b1))

    q = agg16(z2, srcp, dstp, zeros16)
    a2, z3 = _tc_layer(q[0, :N], q[1, :N], a1, dinv, W2, _bb(b2))

    q = agg32(z3, srcp, dstp, zeros32)
    a3, z4 = _tc_layer(q[0, :N], q[1, :N], a2, dinv, W3, _bb(b3))

    q = agg32(z4, srcp, dstp, zeros32)
    a4, z5 = _tc_layer(q[0, :N], q[1, :N], a3, dinv, W4, _bb(b4))

    q = agg64(z5, srcp, dstp, zeros64)
    hlb = _tc_layer5(q[0, :N], q[1, :N], a4, dinv, W5, _bb(b5), Wlb, _bb(blb))

    hpadded = jnp.pad(hlb, ((0, NPAD - N), (0, 0)))
    bpadded = jnp.pad(batch_index, (0, NPAD - N), constant_values=G)
    pmax, psum = _sc_pool(hpadded, bpadded)

    return _tc_final(pmax, psum, batch_index.reshape(1, N),
                     Wla, _bb(bla), Wo, _bb(bo))


# speedup vs baseline: 23.6199x; 1.4326x over previous
import functools

import jax
import jax.numpy as jnp
from jax import lax
from jax.experimental import pallas as pl
from jax.experimental.pallas import tpu as pltpu
from jax.experimental.pallas import tpu_sc as plsc

N = 10000
E = 320000
G = 64
NC = 2
NS = 16
NW = NC * NS
NPAD = 10240
CH = 128
CPT = (E + NW * CH - 1) // (NW * CH)
EPAD = NW * CPT * CH
RPT = NPAD // NS
RPW = NPAD // NW

NBUF = 8

BR = 1000
GRID = N // BR

_P = lax.Precision.HIGHEST
_mesh = plsc.VectorSubcoreMesh(core_axis_name="c", subcore_axis_name="s")
_SC_PARAMS = pltpu.CompilerParams(use_tc_tiling_on_sc=False)


def _make_agg(F):
    @functools.partial(
        pl.kernel,
        out_type=jax.ShapeDtypeStruct((NC, NPAD, F), jnp.float32),
        mesh=_mesh,
        compiler_params=_SC_PARAMS,
        scratch_types=[
            pltpu.VMEM((CPT, CH), jnp.int32),
            pltpu.VMEM((CPT, CH), jnp.int32),
            [pltpu.VMEM((CH, F), jnp.float32) for _ in range(NBUF)],
            [pltpu.SemaphoreType.DMA for _ in range(NBUF)],
            [pltpu.SemaphoreType.DMA for _ in range(NBUF)],
            pltpu.VMEM_SHARED((NPAD, F), jnp.float32),
        ],
    )
    def agg(z_hbm, src_hbm, dst_hbm, zero_hbm, out_hbm, src_v, dst_v, rows_l,
            gsem_l, ssem_l, acc_sh):
        c = lax.axis_index("c")
        s = lax.axis_index("s")
        wid = c * NS + s
        pltpu.sync_copy(zero_hbm.at[pl.ds(s * RPT, RPT)],
                        acc_sh.at[pl.ds(s * RPT, RPT)])
        pltpu.sync_copy(src_hbm.at[wid], src_v)
        pltpu.sync_copy(dst_hbm.at[wid], dst_v)
        plsc.subcore_barrier()

        @pl.loop(0, CPT // NBUF)
        def _(t):
            ghs = [pltpu.async_copy(z_hbm.at[src_v.at[t * NBUF + b]],
                                    rows_l[b], gsem_l[b])
                   for b in range(NBUF)]
            for h in ghs:
                h.wait()
            shs = [pltpu.async_copy(rows_l[b], acc_sh.at[dst_v.at[t * NBUF + b]],
                                    ssem_l[b], add=True)
                   for b in range(NBUF)]
            for h in shs:
                h.wait()

        plsc.subcore_barrier()
        pltpu.sync_copy(acc_sh.at[pl.ds(s * RPT, RPT)],
                        out_hbm.at[c, pl.ds(s * RPT, RPT)])

    return agg


@functools.partial(
    pl.kernel,
    out_type=(jax.ShapeDtypeStruct((NW, G + 1, 128), jnp.float32),
              jax.ShapeDtypeStruct((NW, G + 1, 128), jnp.float32)),
    mesh=_mesh,
    compiler_params=_SC_PARAMS,
    scratch_types=[
        pltpu.VMEM((RPW, 128), jnp.float32),
        pltpu.VMEM((RPW,), jnp.int32),
        pltpu.VMEM((G + 1, 128), jnp.float32),
        pltpu.VMEM((G + 1, 128), jnp.float32),
    ],
)
def _sc_pool(h_hbm, b_hbm, omax_hbm, osum_hbm, rows_v, bid_v, mx_v, sm_v):
    c = lax.axis_index("c")
    s = lax.axis_index("s")
    wid = c * NS + s
    pltpu.sync_copy(h_hbm.at[pl.ds(wid * RPW, RPW)], rows_v)
    pltpu.sync_copy(b_hbm.at[pl.ds(wid * RPW, RPW)], bid_v)

    @pl.loop(0, G + 1)
    def _(g):
        for k in range(8):
            sl = pl.ds(k * 16, 16)
            mx_v[g, sl] = jnp.full((16,), -jnp.inf, jnp.float32)
            sm_v[g, sl] = jnp.zeros((16,), jnp.float32)

    @pl.loop(0, RPW // 16)
    def _(t):
        vb = bid_v[pl.ds(t * 16, 16)]
        for k in range(16):
            bid = vb[k]
            for j in range(8):
                sl = pl.ds(j * 16, 16)
                v = rows_v[t * 16 + k, sl]
                mx_v[bid, sl] = jnp.maximum(mx_v[bid, sl], v)
                sm_v[bid, sl] = sm_v[bid, sl] + v

    pltpu.sync_copy(mx_v, omax_hbm.at[wid])
    pltpu.sync_copy(sm_v, osum_hbm.at[wid])


def _row_spec(f):
    return pl.BlockSpec((BR, f), lambda i: (i, 0))


def _full_spec(shape):
    return pl.BlockSpec(shape, lambda i: tuple(0 for _ in shape))


def _mm(x, w):
    def body(x_ref, w_ref, o_ref):
        o_ref[...] = jnp.dot(x_ref[...], w_ref[...], precision=_P,
                             preferred_element_type=jnp.float32)
    return pl.pallas_call(
        body, grid=(GRID,),
        in_specs=[_row_spec(x.shape[1]), _full_spec(w.shape)],
        out_specs=_row_spec(w.shape[1]),
        out_shape=jax.ShapeDtypeStruct((N, w.shape[1]), jnp.float32),
    )(x, w)


def _tc_dinv_z1(p0, p1, h1):
    def body(p0_ref, p1_ref, h1_ref, dinv_ref, z1_ref):
        deg = 1.0 + p0_ref[:, 0:1] + p1_ref[:, 0:1]
        dinv = 1.0 / jnp.sqrt(deg)
        dinv_ref[...] = dinv
        z1_ref[...] = dinv * h1_ref[...]
    return pl.pallas_call(
        body, grid=(GRID,),
        in_specs=[_row_spec(16), _row_spec(16), _row_spec(16)],
        out_specs=(_row_spec(1), _row_spec(16)),
        out_shape=(jax.ShapeDtypeStruct((N, 1), jnp.float32),
                   jax.ShapeDtypeStruct((N, 16), jnp.float32)),
    )(p0, p1, h1)


def _tc_layer1(q0, q1, h1, dinv, b1):
    def body(q0_ref, q1_ref, h_ref, d_ref, b_ref, a_ref, z_ref):
        dv = d_ref[...]
        u = dv * (q0_ref[...] + q1_ref[...]) + dv * dv * h_ref[...] + b_ref[0:1, :]
        a = jnp.maximum(u, 0.0)
        a_ref[...] = a
        z_ref[...] = dv * a
    return pl.pallas_call(
        body, grid=(GRID,),
        in_specs=[_row_spec(16), _row_spec(16), _row_spec(16), _row_spec(1),
                  _full_spec((8, 16))],
        out_specs=(_row_spec(16), _row_spec(16)),
        out_shape=(jax.ShapeDtypeStruct((N, 16), jnp.float32),
                   jax.ShapeDtypeStruct((N, 16), jnp.float32)),
    )(q0, q1, h1, dinv, b1)


def _tc_layer(q0, q1, ap, dinv, w, b):
    fin = ap.shape[1]
    fout = w.shape[1]

    def body(q0_ref, q1_ref, a_ref, d_ref, w_ref, b_ref, ao_ref, z_ref):
        dv = d_ref[...]
        u = dv * (q0_ref[...] + q1_ref[...]) + dv * dv * a_ref[...]
        a = jnp.maximum(jnp.dot(u, w_ref[...], precision=_P,
                                preferred_element_type=jnp.float32)
                        + b_ref[0:1, :], 0.0)
        ao_ref[...] = a
        z_ref[...] = dv * a
    return pl.pallas_call(
        body, grid=(GRID,),
        in_specs=[_row_spec(fin), _row_spec(fin), _row_spec(fin), _row_spec(1),
                  _full_spec((fin, fout)), _full_spec((8, fout))],
        out_specs=(_row_spec(fout), _row_spec(fout)),
        out_shape=(jax.ShapeDtypeStruct((N, fout), jnp.float32),
                   jax.ShapeDtypeStruct((N, fout), jnp.float32)),
    )(q0, q1, ap, dinv, w, b)


def _tc_layer5(q0, q1, a4, dinv, w5, b5, wlb, blb):
    def body(q0_ref, q1_ref, a_ref, d_ref, w5_ref, b5_ref, wlb_ref, blb_ref, h_ref):
        dv = d_ref[...]
        u = dv * (q0_ref[...] + q1_ref[...]) + dv * dv * a_ref[...]
        a5 = jnp.maximum(jnp.dot(u, w5_ref[...], precision=_P,
                                 preferred_element_type=jnp.float32)
                         + b5_ref[0:1, :], 0.0)
        h_ref[...] = jnp.dot(a5, wlb_ref[...], precision=_P,
                             preferred_element_type=jnp.float32) + blb_ref[0:1, :]
    return pl.pallas_call(
        body, grid=(GRID,),
        in_specs=[_row_spec(64), _row_spec(64), _row_spec(64), _row_spec(1),
                  _full_spec((64, 128)), _full_spec((8, 128)),
                  _full_spec((128, 128)), _full_spec((8, 128))],
        out_specs=_row_spec(128),
        out_shape=jax.ShapeDtypeStruct((N, 128), jnp.float32),
    )(q0, q1, a4, dinv, w5, b5, wlb, blb)


def _tc_final(pmax, psum, batch2d, wla, bla, wo, bo):
    def body(pm_ref, ps_ref, b_ref, wla_ref, bla_ref, wo_ref, bo_ref, o_ref):
        gm = jnp.max(pm_ref[...], axis=0)[:G]
        gs = jnp.sum(ps_ref[...], axis=0)[:G]
        gids = lax.broadcasted_iota(jnp.int32, (G, N), 0)
        mask = (b_ref[...] == gids)
        counts = jnp.sum(mask.astype(jnp.float32), axis=1)
        gm = jnp.where(counts[:, None] > 0, gm, 0.0)
        gmean = gs / jnp.maximum(counts, 1.0)[:, None]
        hc = jnp.concatenate([gm, gmean], axis=1)
        t = jnp.dot(hc, wla_ref[...], precision=_P,
                    preferred_element_type=jnp.float32) + bla_ref[0:1, :]
        o_ref[...] = jnp.dot(t, wo_ref[...], precision=_P,
                             preferred_element_type=jnp.float32) + bo_ref[0:1, :]
    return pl.pallas_call(
        body,
        out_shape=jax.ShapeDtypeStruct((G, 128), jnp.float32),
    )(pmax, psum, batch2d, wla, bla, wo, bo)


def _bb(b):
    return jnp.broadcast_to(b.reshape(1, -1), (8, b.shape[0]))


@jax.jit
def kernel(x, edge_index, batch_index, W1, b1, W2, b2, W3, b3, W4, b4, W5, b5,
           Wlb, blb, Wla, bla, Wo, bo):
    src = edge_index[0]
    dst = edge_index[1]
    epad = EPAD - E
    srcp = jnp.concatenate([src, jnp.zeros((epad,), jnp.int32)]).reshape(NW, CPT, CH)
    dstp = jnp.concatenate([dst, jnp.full((epad,), NPAD - 1, jnp.int32)]).reshape(NW, CPT, CH)

    zeros16 = jnp.zeros((NPAD, 16), jnp.float32)
    zeros32 = jnp.zeros((NPAD, 32), jnp.float32)
    zeros64 = jnp.zeros((NPAD, 64), jnp.float32)
    ones16 = jnp.ones((N, 16), jnp.float32)

    agg16 = _make_agg(16)
    agg32 = _make_agg(32)
    agg64 = _make_agg(64)

    pdeg = agg16(ones16, srcp, dstp, zeros16)
    h1 = _mm(x, W1)
    dinv, z1 = _tc_dinv_z1(pdeg[0, :N], pdeg[1, :N], h1)

    q = agg16(z1, srcp, dstp, zeros16)
    a1, z2 = _tc_layer1(q[0, :N], q[1, :N], h1, dinv, _bb(b1))

    q = agg16(z2, srcp, dstp, zeros16)
    a2, z3 = _tc_layer(q[0, :N], q[1, :N], a1, dinv, W2, _bb(b2))

    q = agg32(z3, srcp, dstp, zeros32)
    a3, z4 = _tc_layer(q[0, :N], q[1, :N], a2, dinv, W3, _bb(b3))

    q = agg32(z4, srcp, dstp, zeros32)
    a4, z5 = _tc_layer(q[0, :N], q[1, :N], a3, dinv, W4, _bb(b4))

    q = agg64(z5, srcp, dstp, zeros64)
    hlb = _tc_layer5(q[0, :N], q[1, :N], a4, dinv, W5, _bb(b5), Wlb, _bb(blb))

    hpadded = jnp.pad(hlb, ((0, NPAD - N), (0, 0)))
    bpadded = jnp.pad(batch_index, (0, NPAD - N), constant_values=G)
    pmax, psum = _sc_pool(hpadded, bpadded)

    return _tc_final(pmax, psum, batch_index.reshape(1, N),
                     Wla, _bb(bla), Wo, _bb(bo))


# speedup vs baseline: 25.0364x; 1.0600x over previous
import functools

import jax
import jax.numpy as jnp
from jax import lax
from jax.experimental import pallas as pl
from jax.experimental.pallas import tpu as pltpu
from jax.experimental.pallas import tpu_sc as plsc

N = 10000
E = 320000
G = 64
NC = 2
NS = 16
NW = NC * NS
NPAD = 10240
CH = 128
CPT = (E + NW * CH - 1) // (NW * CH)
EPAD = NW * CPT * CH
RPT = NPAD // NS
RPW = NPAD // NW

NBUF = 8

BR = 1000
GRID = N // BR

_P = lax.Precision.HIGHEST
_mesh = plsc.VectorSubcoreMesh(core_axis_name="c", subcore_axis_name="s")
_SC_PARAMS = pltpu.CompilerParams(use_tc_tiling_on_sc=False)


def _make_agg(F):
    @functools.partial(
        pl.kernel,
        out_type=jax.ShapeDtypeStruct((NC, NPAD, F), jnp.float32),
        mesh=_mesh,
        compiler_params=_SC_PARAMS,
        scratch_types=[
            pltpu.VMEM((CPT, CH), jnp.int32),
            pltpu.VMEM((CPT, CH), jnp.int32),
            [pltpu.VMEM((CH, F), jnp.float32) for _ in range(NBUF)],
            [pltpu.SemaphoreType.DMA for _ in range(NBUF)],
            [pltpu.SemaphoreType.DMA for _ in range(NBUF)],
            pltpu.VMEM_SHARED((NPAD, F), jnp.float32),
        ],
    )
    def agg(z_hbm, src_hbm, dst_hbm, zero_hbm, out_hbm, src_v, dst_v, rows_l,
            gsem_l, ssem_l, acc_sh):
        c = lax.axis_index("c")
        s = lax.axis_index("s")
        wid = c * NS + s
        pltpu.sync_copy(zero_hbm.at[pl.ds(s * RPT, RPT)],
                        acc_sh.at[pl.ds(s * RPT, RPT)])
        pltpu.sync_copy(src_hbm.at[wid], src_v)
        pltpu.sync_copy(dst_hbm.at[wid], dst_v)
        plsc.subcore_barrier()

        @pl.loop(0, CPT // NBUF)
        def _(t):
            ghs = [pltpu.async_copy(z_hbm.at[src_v.at[t * NBUF + b]],
                                    rows_l[b], gsem_l[b])
                   for b in range(NBUF)]
            shs = []
            for b in range(NBUF):
                ghs[b].wait()
                shs.append(pltpu.async_copy(
                    rows_l[b], acc_sh.at[dst_v.at[t * NBUF + b]],
                    ssem_l[b], add=True))
            for h in shs:
                h.wait()

        plsc.subcore_barrier()
        pltpu.sync_copy(acc_sh.at[pl.ds(s * RPT, RPT)],
                        out_hbm.at[c, pl.ds(s * RPT, RPT)])

    return agg


@functools.partial(
    pl.kernel,
    out_type=jax.ShapeDtypeStruct((NC, NPAD, 16), jnp.float32),
    mesh=_mesh,
    compiler_params=_SC_PARAMS,
    scratch_types=[
        pltpu.VMEM((CPT, CH), jnp.int32),
        pltpu.VMEM((CH, 16), jnp.float32),
        [pltpu.SemaphoreType.DMA for _ in range(NBUF)],
        pltpu.VMEM_SHARED((NPAD, 16), jnp.float32),
    ],
)
def _sc_deg(ones_hbm, dst_hbm, zero_hbm, out_hbm, dst_v, ones_v, ssem_l, acc_sh):
    c = lax.axis_index("c")
    s = lax.axis_index("s")
    wid = c * NS + s
    pltpu.sync_copy(zero_hbm.at[pl.ds(s * RPT, RPT)],
                    acc_sh.at[pl.ds(s * RPT, RPT)])
    pltpu.sync_copy(dst_hbm.at[wid], dst_v)
    pltpu.sync_copy(ones_hbm, ones_v)
    plsc.subcore_barrier()

    @pl.loop(0, CPT // NBUF)
    def _(t):
        shs = [pltpu.async_copy(ones_v, acc_sh.at[dst_v.at[t * NBUF + b]],
                                ssem_l[b], add=True)
               for b in range(NBUF)]
        for h in shs:
            h.wait()

    plsc.subcore_barrier()
    pltpu.sync_copy(acc_sh.at[pl.ds(s * RPT, RPT)],
                    out_hbm.at[c, pl.ds(s * RPT, RPT)])


@functools.partial(
    pl.kernel,
    out_type=(jax.ShapeDtypeStruct((NW, G + 1, 128), jnp.float32),
              jax.ShapeDtypeStruct((NW, G + 1, 128), jnp.float32)),
    mesh=_mesh,
    compiler_params=_SC_PARAMS,
    scratch_types=[
        pltpu.VMEM((RPW, 128), jnp.float32),
        pltpu.VMEM((RPW,), jnp.int32),
        pltpu.VMEM((G + 1, 128), jnp.float32),
        pltpu.VMEM((G + 1, 128), jnp.float32),
    ],
)
def _sc_pool(h_hbm, b_hbm, omax_hbm, osum_hbm, rows_v, bid_v, mx_v, sm_v):
    c = lax.axis_index("c")
    s = lax.axis_index("s")
    wid = c * NS + s
    pltpu.sync_copy(h_hbm.at[pl.ds(wid * RPW, RPW)], rows_v)
    pltpu.sync_copy(b_hbm.at[pl.ds(wid * RPW, RPW)], bid_v)

    @pl.loop(0, G + 1)
    def _(g):
        for k in range(8):
            sl = pl.ds(k * 16, 16)
            mx_v[g, sl] = jnp.full((16,), -jnp.inf, jnp.float32)
            sm_v[g, sl] = jnp.zeros((16,), jnp.float32)

    @pl.loop(0, RPW // 16)
    def _(t):
        vb = bid_v[pl.ds(t * 16, 16)]
        for k in range(16):
            bid = vb[k]
            for j in range(8):
                sl = pl.ds(j * 16, 16)
                v = rows_v[t * 16 + k, sl]
                mx_v[bid, sl] = jnp.maximum(mx_v[bid, sl], v)
                sm_v[bid, sl] = sm_v[bid, sl] + v

    pltpu.sync_copy(mx_v, omax_hbm.at[wid])
    pltpu.sync_copy(sm_v, osum_hbm.at[wid])


def _row_spec(f):
    return pl.BlockSpec((BR, f), lambda i: (i, 0))


def _full_spec(shape):
    return pl.BlockSpec(shape, lambda i: tuple(0 for _ in shape))


def _mm(x, w):
    def body(x_ref, w_ref, o_ref):
        o_ref[...] = jnp.dot(x_ref[...], w_ref[...], precision=_P,
                             preferred_element_type=jnp.float32)
    return pl.pallas_call(
        body, grid=(GRID,),
        in_specs=[_row_spec(x.shape[1]), _full_spec(w.shape)],
        out_specs=_row_spec(w.shape[1]),
        out_shape=jax.ShapeDtypeStruct((N, w.shape[1]), jnp.float32),
    )(x, w)


def _tc_dinv_z1(p0, p1, h1):
    def body(p0_ref, p1_ref, h1_ref, dinv_ref, z1_ref):
        deg = 1.0 + p0_ref[:, 0:1] + p1_ref[:, 0:1]
        dinv = 1.0 / jnp.sqrt(deg)
        dinv_ref[...] = dinv
        z1_ref[...] = dinv * h1_ref[...]
    return pl.pallas_call(
        body, grid=(GRID,),
        in_specs=[_row_spec(16), _row_spec(16), _row_spec(16)],
        out_specs=(_row_spec(1), _row_spec(16)),
        out_shape=(jax.ShapeDtypeStruct((N, 1), jnp.float32),
                   jax.ShapeDtypeStruct((N, 16), jnp.float32)),
    )(p0, p1, h1)


def _tc_layer1(q0, q1, h1, dinv, b1):
    def body(q0_ref, q1_ref, h_ref, d_ref, b_ref, a_ref, z_ref):
        dv = d_ref[...]
        u = dv * (q0_ref[...] + q1_ref[...]) + dv * dv * h_ref[...] + b_ref[0:1, :]
        a = jnp.maximum(u, 0.0)
        a_ref[...] = a
        z_ref[...] = dv * a
    return pl.pallas_call(
        body, grid=(GRID,),
        in_specs=[_row_spec(16), _row_spec(16), _row_spec(16), _row_spec(1),
                  _full_spec((8, 16))],
        out_specs=(_row_spec(16), _row_spec(16)),
        out_shape=(jax.ShapeDtypeStruct((N, 16), jnp.float32),
                   jax.ShapeDtypeStruct((N, 16), jnp.float32)),
    )(q0, q1, h1, dinv, b1)


def _tc_layer(q0, q1, ap, dinv, w, b):
    fin = ap.shape[1]
    fout = w.shape[1]

    def body(q0_ref, q1_ref, a_ref, d_ref, w_ref, b_ref, ao_ref, z_ref):
        dv = d_ref[...]
        u = dv * (q0_ref[...] + q1_ref[...]) + dv * dv * a_ref[...]
        a = jnp.maximum(jnp.dot(u, w_ref[...], precision=_P,
                                preferred_element_type=jnp.float32)
                        + b_ref[0:1, :], 0.0)
        ao_ref[...] = a
        z_ref[...] = dv * a
    return pl.pallas_call(
        body, grid=(GRID,),
        in_specs=[_row_spec(fin), _row_spec(fin), _row_spec(fin), _row_spec(1),
                  _full_spec((fin, fout)), _full_spec((8, fout))],
        out_specs=(_row_spec(fout), _row_spec(fout)),
        out_shape=(jax.ShapeDtypeStruct((N, fout), jnp.float32),
                   jax.ShapeDtypeStruct((N, fout), jnp.float32)),
    )(q0, q1, ap, dinv, w, b)


def _tc_layer5(q0, q1, a4, dinv, w5, b5, wlb, blb):
    def body(q0_ref, q1_ref, a_ref, d_ref, w5_ref, b5_ref, wlb_ref, blb_ref, h_ref):
        dv = d_ref[...]
        u = dv * (q0_ref[...] + q1_ref[...]) + dv * dv * a_ref[...]
        a5 = jnp.maximum(jnp.dot(u, w5_ref[...], precision=_P,
                                 preferred_element_type=jnp.float32)
                         + b5_ref[0:1, :], 0.0)
        h_ref[...] = jnp.dot(a5, wlb_ref[...], precision=_P,
                             preferred_element_type=jnp.float32) + blb_ref[0:1, :]
    return pl.pallas_call(
        body, grid=(GRID,),
        in_specs=[_row_spec(64), _row_spec(64), _row_spec(64), _row_spec(1),
                  _full_spec((64, 128)), _full_spec((8, 128)),
                  _full_spec((128, 128)), _full_spec((8, 128))],
        out_specs=_row_spec(128),
        out_shape=jax.ShapeDtypeStruct((N, 128), jnp.float32),
    )(q0, q1, a4, dinv, w5, b5, wlb, blb)


def _tc_final(pmax, psum, batch2d, wla, bla, wo, bo):
    def body(pm_ref, ps_ref, b_ref, wla_ref, bla_ref, wo_ref, bo_ref, o_ref):
        gm = jnp.max(pm_ref[...], axis=0)[:G]
        gs = jnp.sum(ps_ref[...], axis=0)[:G]
        gids = lax.broadcasted_iota(jnp.int32, (G, N), 0)
        mask = (b_ref[...] == gids)
        counts = jnp.sum(mask.astype(jnp.float32), axis=1)
        gm = jnp.where(counts[:, None] > 0, gm, 0.0)
        gmean = gs / jnp.maximum(counts, 1.0)[:, None]
        hc = jnp.concatenate([gm, gmean], axis=1)
        t = jnp.dot(hc, wla_ref[...], precision=_P,
                    preferred_element_type=jnp.float32) + bla_ref[0:1, :]
        o_ref[...] = jnp.dot(t, wo_ref[...], precision=_P,
                             preferred_element_type=jnp.float32) + bo_ref[0:1, :]
    return pl.pallas_call(
        body,
        out_shape=jax.ShapeDtypeStruct((G, 128), jnp.float32),
    )(pmax, psum, batch2d, wla, bla, wo, bo)


def _bb(b):
    return jnp.broadcast_to(b.reshape(1, -1), (8, b.shape[0]))


@jax.jit
def kernel(x, edge_index, batch_index, W1, b1, W2, b2, W3, b3, W4, b4, W5, b5,
           Wlb, blb, Wla, bla, Wo, bo):
    src = edge_index[0]
    dst = edge_index[1]
    epad = EPAD - E
    srcp = jnp.concatenate([src, jnp.zeros((epad,), jnp.int32)]).reshape(NW, CPT, CH)
    dstp = jnp.concatenate([dst, jnp.full((epad,), NPAD - 1, jnp.int32)]).reshape(NW, CPT, CH)

    zeros16 = jnp.zeros((NPAD, 16), jnp.float32)
    zeros32 = jnp.zeros((NPAD, 32), jnp.float32)
    zeros64 = jnp.zeros((NPAD, 64), jnp.float32)
    ones16 = jnp.ones((CH, 16), jnp.float32)

    agg16 = _make_agg(16)
    agg32 = _make_agg(32)
    agg64 = _make_agg(64)

    pdeg = _sc_deg(ones16, dstp, zeros16)
    h1 = _mm(x, W1)
    dinv, z1 = _tc_dinv_z1(pdeg[0, :N], pdeg[1, :N], h1)

    q = agg16(z1, srcp, dstp, zeros16)
    a1, z2 = _tc_layer1(q[0, :N], q[1, :N], h1, dinv, _bb(b1))

    q = agg16(z2, srcp, dstp, zeros16)
    a2, z3 = _tc_layer(q[0, :N], q[1, :N], a1, dinv, W2, _bb(b2))

    q = agg32(z3, srcp, dstp, zeros32)
    a3, z4 = _tc_layer(q[0, :N], q[1, :N], a2, dinv, W3, _bb(b3))

    q = agg32(z4, srcp, dstp, zeros32)
    a4, z5 = _tc_layer(q[0, :N], q[1, :N], a3, dinv, W4, _bb(b4))

    q = agg64(z5, srcp, dstp, zeros64)
    hlb = _tc_layer5(q[0, :N], q[1, :N], a4, dinv, W5, _bb(b5), Wlb, _bb(blb))

    hpadded = jnp.pad(hlb, ((0, NPAD - N), (0, 0)))
    bpadded = jnp.pad(batch_index, (0, NPAD - N), constant_values=G)
    pmax, psum = _sc_pool(hpadded, bpadded)

    return _tc_final(pmax, psum, batch_index.reshape(1, N),
                     Wla, _bb(bla), Wo, _bb(bo))
